# baseline (speedup 1.0000x reference)
import jax
import jax.numpy as jnp
from jax.experimental import pallas as pl
from jax.experimental.pallas import tpu as pltpu

MAX_CTX = 32768
DIM = 2048

_HALF_ROWS = MAX_CTX // 2
_NCHUNK = 8
_CROWS = _HALF_ROWS // _NCHUNK


def _dma_body(mem_ref, h_ref, out_ref, sems):
    copies = []
    for k in range(_NCHUNK):
        copies.append(pltpu.make_async_copy(
            mem_ref.at[pl.ds(_HALF_ROWS + k * _CROWS, _CROWS), :],
            out_ref.at[pl.ds(k * _CROWS, _CROWS), :],
            sems.at[2 * k]))
        copies.append(pltpu.make_async_copy(
            h_ref.at[pl.ds(k * _CROWS, _CROWS), :],
            out_ref.at[pl.ds(_HALF_ROWS + k * _CROWS, _CROWS), :],
            sems.at[2 * k + 1]))
    for c in copies:
        c.start()
    for c in copies:
        c.wait()


def kernel(h, mem):
    B, L, D = h.shape
    flat = h.reshape(B * L, D)
    new_mem = pl.pallas_call(
        _dma_body,
        in_specs=[
            pl.BlockSpec(memory_space=pltpu.MemorySpace.HBM),
            pl.BlockSpec(memory_space=pltpu.MemorySpace.HBM),
        ],
        out_specs=pl.BlockSpec(memory_space=pltpu.MemorySpace.HBM),
        out_shape=jax.ShapeDtypeStruct((MAX_CTX, D), h.dtype),
        scratch_shapes=[pltpu.SemaphoreType.DMA((2 * _NCHUNK,))],
    )(mem, flat)
    return h, new_mem


# speedup vs baseline: 33.0380x; 33.0380x over previous
import jax
import jax.numpy as jnp
from jax.experimental import pallas as pl
from jax.experimental.pallas import tpu as pltpu

MAX_CTX = 32768
DIM = 2048

_HALF_ROWS = MAX_CTX // 2
_BR = 512
_NC = MAX_CTX // _BR
_K = 8
_W = 4


def _src_slice(mem_ref, h_ref, c):
    row = c * _BR
    if row < _HALF_ROWS:
        return mem_ref.at[pl.ds(_HALF_ROWS + row, _BR), :]
    return h_ref.at[pl.ds(row - _HALF_ROWS, _BR), :]


def _stream_body(mem_ref, h_ref, out_ref, vbuf, rsem, wsem):
    def read(c):
        return pltpu.make_async_copy(
            _src_slice(mem_ref, h_ref, c), vbuf.at[c % _K], rsem.at[c % _K])

    def write(c):
        return pltpu.make_async_copy(
            vbuf.at[c % _K], out_ref.at[pl.ds(c * _BR, _BR), :], wsem.at[c % _K])

    for b in range(_K):
        read(b).start()
    waited_w = 0
    for c in range(_NC):
        read(c).wait()
        write(c).start()
        t = c - _W
        if t >= 0 and t + _K < _NC:
            write(t).wait()
            waited_w = t + 1
            read(t + _K).start()
    for c in range(waited_w, _NC):
        write(c).wait()


def kernel(h, mem):
    B, L, D = h.shape
    flat = h.reshape(B * L, D)
    new_mem = pl.pallas_call(
        _stream_body,
        in_specs=[
            pl.BlockSpec(memory_space=pltpu.MemorySpace.HBM),
            pl.BlockSpec(memory_space=pltpu.MemorySpace.HBM),
        ],
        out_specs=pl.BlockSpec(memory_space=pltpu.MemorySpace.HBM),
        out_shape=jax.ShapeDtypeStruct((MAX_CTX, D), h.dtype),
        scratch_shapes=[
            pltpu.VMEM((_K, _BR, DIM), jnp.float32),
            pltpu.SemaphoreType.DMA((_K,)),
            pltpu.SemaphoreType.DMA((_K,)),
        ],
    )(mem, flat)
    return h, new_mem
